# baseline (speedup 1.0000x reference)
import functools

import jax
import jax.numpy as jnp
from jax import lax
from jax.experimental import pallas as pl
from jax.experimental.pallas import tpu as pltpu
from jax.experimental.pallas import tpu_sc as plsc

EPAD = 128


def _make_sc_gather(V, D, N):
    info = plsc.get_sparse_core_info()
    NC, NS = info.num_cores, info.num_subcores
    NW = NC * NS
    per_w = N // NW
    CH = 128
    n_ch = per_w // CH
    mesh = plsc.VectorSubcoreMesh(core_axis_name="c", subcore_axis_name="s")

    NB = 4

    @functools.partial(
        pl.kernel,
        mesh=mesh,
        out_type=jax.ShapeDtypeStruct((N, D), jnp.float32),
        scratch_types=(
            [pltpu.VMEM((n_ch, CH), jnp.int32)]
            + [pltpu.VMEM((CH, D), jnp.float32)] * NB
            + [pltpu.SemaphoreType.DMA] * (2 * NB)
        ),
    )
    def gather(table_hbm, idx_hbm, out_hbm, idx_v, *bufs_sems):
        bufs = bufs_sems[:NB]
        gsems = bufs_sems[NB:2 * NB]
        ssems = bufs_sems[2 * NB:]
        wid = lax.axis_index("s") * NC + lax.axis_index("c")
        base = wid * per_w
        pltpu.sync_copy(idx_hbm.at[pl.ds(wid * n_ch, n_ch)], idx_v)
        gcp = [None] * n_ch
        scp = [None] * n_ch
        for j in range(n_ch):
            b = j % NB
            if j >= NB:
                scp[j - NB].wait()
            gcp[j] = pltpu.async_copy(
                table_hbm.at[idx_v.at[j]], bufs[b], gsems[b])
            if j >= 1:
                gcp[j - 1].wait()
                scp[j - 1] = pltpu.async_copy(
                    bufs[(j - 1) % NB],
                    out_hbm.at[pl.ds(base + (j - 1) * CH, CH)],
                    ssems[(j - 1) % NB])
        last = n_ch - 1
        gcp[last].wait()
        scp[last] = pltpu.async_copy(
            bufs[last % NB], out_hbm.at[pl.ds(base + last * CH, CH)],
            ssems[last % NB])
        for j in range(max(0, n_ch - NB + 1), n_ch):
            scp[j].wait()

    return gather


def _pad_body(src_ref, dst_ref):
    E = src_ref.shape[1]
    dst_ref[:, 0:E] = src_ref[...]
    dst_ref[:, E:] = jnp.zeros_like(dst_ref[:, E:])


def _pad_table(emb, EP):
    V, E = emb.shape
    RB = 2000
    return pl.pallas_call(
        _pad_body,
        grid=(V // RB,),
        in_specs=[pl.BlockSpec((RB, E), lambda i: (i, 0))],
        out_specs=pl.BlockSpec((RB, EP), lambda i: (i, 0)),
        out_shape=jax.ShapeDtypeStruct((V, EP), jnp.float32),
    )(emb)


UNROLL = 32


def _lstm_body(x_ref, wxf_ref, wxb_ref, wh_ref, bias_ref, wout_ref,
               bout_ref, out_ref, h_ref, c_ref):
    L = out_ref.shape[0]
    H2 = c_ref.shape[1]
    s = pl.program_id(0)

    @pl.when(s == 0)
    def _init():
        out_ref[...] = jnp.zeros_like(out_ref)
        h_ref[...] = jnp.zeros_like(h_ref)
        c_ref[...] = jnp.zeros_like(c_ref)

    h = h_ref[...]
    c = c_ref[...]
    T = bout_ref.shape[0]
    for k in range(UNROLL):
        t = s * UNROLL + k
        rt = (L - 1) - t
        g = (jnp.dot(x_ref[t].astype(jnp.bfloat16), wxf_ref[...],
                     preferred_element_type=jnp.float32)
             + jnp.dot(x_ref[rt].astype(jnp.bfloat16), wxb_ref[...],
                       preferred_element_type=jnp.float32)
             + bias_ref[...]
             + jnp.dot(h, wh_ref[...], preferred_element_type=jnp.float32))
        tg = jnp.tanh(g)
        gi = 0.5 * tg[:, 0:H2] + 0.5
        gf = 0.5 * tg[:, H2:2 * H2] + 0.5
        gg = tg[:, 2 * H2:3 * H2]
        go = 0.5 * tg[:, 3 * H2:4 * H2] + 0.5
        c = gf * c + gi * gg
        hf32 = go * jnp.tanh(c)
        h = hf32.astype(jnp.bfloat16)
        p = jnp.dot(hf32, wout_ref[...], preferred_element_type=jnp.float32)
        pt = jnp.swapaxes(p, 0, 1)
        out_ref[t] = out_ref[t] + pt[0:T] + bout_ref[...]
        out_ref[rt] = out_ref[rt] + pt[T:2 * T]
    h_ref[...] = h
    c_ref[...] = c


def _scatter_gates(wt, fwd):
    blocks = jnp.split(wt, 4, axis=1)
    z = jnp.zeros_like(blocks[0])
    cols = []
    for b in blocks:
        cols += ([b, z] if fwd else [z, b])
    return jnp.concatenate(cols, axis=1)


def kernel(inp, emb, w_ih_f, w_hh_f, b_ih_f, b_hh_f,
           w_ih_b, w_hh_b, b_ih_b, b_hh_b, W_out, b_out):
    B, L = inp.shape
    V, E = emb.shape
    H4 = w_ih_f.shape[0]
    H = H4 // 4
    T = W_out.shape[0]

    N = B * L
    CH = 128
    idx2 = jnp.transpose(inp).astype(jnp.int32).reshape(N // CH, CH)

    table = _pad_table(emb, EPAD)

    xg = _make_sc_gather(V, EPAD, N)(table, idx2)
    x = xg.reshape(L, B, EPAD)

    def padE(w):
        return jnp.pad(w, ((0, EPAD - E), (0, 0)))

    gscale = jnp.full((8 * H,), 0.5, jnp.float32).at[4 * H:6 * H].set(1.0)
    wxf = (_scatter_gates(padE(w_ih_f.T), True) * gscale).astype(jnp.bfloat16)
    wxb = (_scatter_gates(padE(w_ih_b.T), False) * gscale).astype(jnp.bfloat16)
    wh = (jnp.concatenate([
        _scatter_gates(w_hh_f.T, True),
        _scatter_gates(w_hh_b.T, False),
    ], axis=0) * gscale).astype(jnp.bfloat16)
    bias = ((_scatter_gates((b_ih_f + b_hh_f)[None, :], True)
             + _scatter_gates((b_ih_b + b_hh_b)[None, :], False))
            * gscale)

    wout = jnp.zeros((2 * H, 2 * T), jnp.float32)
    wout = wout.at[:H, :T].set(W_out[:, :H].T)
    wout = wout.at[H:, T:].set(W_out[:, H:].T)
    bout = b_out[:, None]

    out = pl.pallas_call(
        _lstm_body,
        grid=(L // UNROLL,),
        in_specs=[
            pl.BlockSpec((L, B, EPAD), lambda t: (0, 0, 0)),
            pl.BlockSpec((EPAD, 8 * H), lambda t: (0, 0)),
            pl.BlockSpec((EPAD, 8 * H), lambda t: (0, 0)),
            pl.BlockSpec((2 * H, 8 * H), lambda t: (0, 0)),
            pl.BlockSpec((1, 8 * H), lambda t: (0, 0)),
            pl.BlockSpec((2 * H, 2 * T), lambda t: (0, 0)),
            pl.BlockSpec((T, 1), lambda t: (0, 0)),
        ],
        out_specs=pl.BlockSpec((L, T, B), lambda t: (0, 0, 0)),
        out_shape=jax.ShapeDtypeStruct((L, T, B), jnp.float32),
        scratch_shapes=[
            pltpu.VMEM((B, 2 * H), jnp.bfloat16),
            pltpu.VMEM((B, 2 * H), jnp.float32),
        ],
        compiler_params=pltpu.CompilerParams(
            dimension_semantics=("arbitrary",)),
    )(x, wxf, wxb, wh, bias, wout, bout)

    return jnp.transpose(out, (2, 0, 1))


# speedup vs baseline: 1.0941x; 1.0941x over previous
import functools

import jax
import jax.numpy as jnp
from jax import lax
from jax.experimental import pallas as pl
from jax.experimental.pallas import tpu as pltpu
from jax.experimental.pallas import tpu_sc as plsc

EPAD = 128


def _make_sc_gather(V, D, N):
    info = plsc.get_sparse_core_info()
    NC, NS = info.num_cores, info.num_subcores
    NW = NC * NS
    per_w = N // NW
    CH = 128
    n_ch = per_w // CH
    mesh = plsc.VectorSubcoreMesh(core_axis_name="c", subcore_axis_name="s")

    NB = 4

    @functools.partial(
        pl.kernel,
        mesh=mesh,
        out_type=jax.ShapeDtypeStruct((N, D), jnp.float32),
        scratch_types=(
            [pltpu.VMEM((n_ch, CH), jnp.int32)]
            + [pltpu.VMEM((CH, D), jnp.float32)] * NB
            + [pltpu.SemaphoreType.DMA] * (2 * NB)
        ),
    )
    def gather(table_hbm, idx_hbm, out_hbm, idx_v, *bufs_sems):
        bufs = bufs_sems[:NB]
        gsems = bufs_sems[NB:2 * NB]
        ssems = bufs_sems[2 * NB:]
        wid = lax.axis_index("s") * NC + lax.axis_index("c")
        base = wid * per_w
        pltpu.sync_copy(idx_hbm.at[pl.ds(wid * n_ch, n_ch)], idx_v)
        gcp = [None] * n_ch
        scp = [None] * n_ch
        for j in range(n_ch):
            b = j % NB
            if j >= NB:
                scp[j - NB].wait()
            gcp[j] = pltpu.async_copy(
                table_hbm.at[idx_v.at[j]], bufs[b], gsems[b])
            if j >= 1:
                gcp[j - 1].wait()
                scp[j - 1] = pltpu.async_copy(
                    bufs[(j - 1) % NB],
                    out_hbm.at[pl.ds(base + (j - 1) * CH, CH)],
                    ssems[(j - 1) % NB])
        last = n_ch - 1
        gcp[last].wait()
        scp[last] = pltpu.async_copy(
            bufs[last % NB], out_hbm.at[pl.ds(base + last * CH, CH)],
            ssems[last % NB])
        for j in range(max(0, n_ch - NB + 1), n_ch):
            scp[j].wait()

    return gather


def _pad_body(src_ref, dst_ref):
    E = src_ref.shape[1]
    dst_ref[:, 0:E] = src_ref[...]
    dst_ref[:, E:] = jnp.zeros_like(dst_ref[:, E:])


def _pad_table(emb, EP):
    V, E = emb.shape
    RB = 10000
    return pl.pallas_call(
        _pad_body,
        grid=(V // RB,),
        in_specs=[pl.BlockSpec((RB, E), lambda i: (i, 0))],
        out_specs=pl.BlockSpec((RB, EP), lambda i: (i, 0)),
        out_shape=jax.ShapeDtypeStruct((V, EP), jnp.float32),
    )(emb)


UNROLL = 32


def _lstm_body(x_ref, wxf_ref, wxb_ref, wh_ref, bias_ref, wout_ref,
               bout_ref, out_ref, h_ref, c_ref):
    L = out_ref.shape[0]
    H2 = c_ref.shape[1]
    s = pl.program_id(0)

    @pl.when(s == 0)
    def _init():
        out_ref[...] = jnp.zeros_like(out_ref)
        h_ref[...] = jnp.zeros_like(h_ref)
        c_ref[...] = jnp.zeros_like(c_ref)

    h = h_ref[...]
    c = c_ref[...]
    T = bout_ref.shape[0]
    for k in range(UNROLL):
        t = s * UNROLL + k
        rt = (L - 1) - t
        g = (jnp.dot(x_ref[t].astype(jnp.bfloat16), wxf_ref[...],
                     preferred_element_type=jnp.float32)
             + jnp.dot(x_ref[rt].astype(jnp.bfloat16), wxb_ref[...],
                       preferred_element_type=jnp.float32)
             + bias_ref[...]
             + jnp.dot(h, wh_ref[...], preferred_element_type=jnp.float32))
        tg = jnp.tanh(g)
        gi = 0.5 * tg[:, 0:H2] + 0.5
        gf = 0.5 * tg[:, H2:2 * H2] + 0.5
        gg = tg[:, 2 * H2:3 * H2]
        go = 0.5 * tg[:, 3 * H2:4 * H2] + 0.5
        c = gf * c + gi * gg
        hf32 = go * jnp.tanh(c)
        h = hf32.astype(jnp.bfloat16)
        p = jnp.dot(hf32, wout_ref[...], preferred_element_type=jnp.float32)
        pt = jnp.swapaxes(p, 0, 1)
        out_ref[t] = out_ref[t] + pt[0:T] + bout_ref[...]
        out_ref[rt] = out_ref[rt] + pt[T:2 * T]
    h_ref[...] = h
    c_ref[...] = c


def _scatter_gates(wt, fwd):
    blocks = jnp.split(wt, 4, axis=1)
    z = jnp.zeros_like(blocks[0])
    cols = []
    for b in blocks:
        cols += ([b, z] if fwd else [z, b])
    return jnp.concatenate(cols, axis=1)


def kernel(inp, emb, w_ih_f, w_hh_f, b_ih_f, b_hh_f,
           w_ih_b, w_hh_b, b_ih_b, b_hh_b, W_out, b_out):
    B, L = inp.shape
    V, E = emb.shape
    H4 = w_ih_f.shape[0]
    H = H4 // 4
    T = W_out.shape[0]

    N = B * L
    CH = 128
    idx2 = jnp.transpose(inp).astype(jnp.int32).reshape(N // CH, CH)

    table = _pad_table(emb, EPAD)

    xg = _make_sc_gather(V, EPAD, N)(table, idx2)
    x = xg.reshape(L, B, EPAD)

    def padE(w):
        return jnp.pad(w, ((0, EPAD - E), (0, 0)))

    gscale = jnp.full((8 * H,), 0.5, jnp.float32).at[4 * H:6 * H].set(1.0)
    wxf = (_scatter_gates(padE(w_ih_f.T), True) * gscale).astype(jnp.bfloat16)
    wxb = (_scatter_gates(padE(w_ih_b.T), False) * gscale).astype(jnp.bfloat16)
    wh = (jnp.concatenate([
        _scatter_gates(w_hh_f.T, True),
        _scatter_gates(w_hh_b.T, False),
    ], axis=0) * gscale).astype(jnp.bfloat16)
    bias = ((_scatter_gates((b_ih_f + b_hh_f)[None, :], True)
             + _scatter_gates((b_ih_b + b_hh_b)[None, :], False))
            * gscale)

    wout = jnp.zeros((2 * H, 2 * T), jnp.float32)
    wout = wout.at[:H, :T].set(W_out[:, :H].T)
    wout = wout.at[H:, T:].set(W_out[:, H:].T)
    bout = b_out[:, None]

    out = pl.pallas_call(
        _lstm_body,
        grid=(L // UNROLL,),
        in_specs=[
            pl.BlockSpec((L, B, EPAD), lambda t: (0, 0, 0)),
            pl.BlockSpec((EPAD, 8 * H), lambda t: (0, 0)),
            pl.BlockSpec((EPAD, 8 * H), lambda t: (0, 0)),
            pl.BlockSpec((2 * H, 8 * H), lambda t: (0, 0)),
            pl.BlockSpec((1, 8 * H), lambda t: (0, 0)),
            pl.BlockSpec((2 * H, 2 * T), lambda t: (0, 0)),
            pl.BlockSpec((T, 1), lambda t: (0, 0)),
        ],
        out_specs=pl.BlockSpec((L, T, B), lambda t: (0, 0, 0)),
        out_shape=jax.ShapeDtypeStruct((L, T, B), jnp.float32),
        scratch_shapes=[
            pltpu.VMEM((B, 2 * H), jnp.bfloat16),
            pltpu.VMEM((B, 2 * H), jnp.float32),
        ],
        compiler_params=pltpu.CompilerParams(
            dimension_semantics=("arbitrary",)),
    )(x, wxf, wxb, wh, bias, wout, bout)

    return jnp.transpose(out, (2, 0, 1))


# speedup vs baseline: 1.1014x; 1.0067x over previous
import functools

import jax
import jax.numpy as jnp
from jax import lax
from jax.experimental import pallas as pl
from jax.experimental.pallas import tpu as pltpu
from jax.experimental.pallas import tpu_sc as plsc

EPAD = 128


def _make_sc_gather(V, D, N):
    info = plsc.get_sparse_core_info()
    NC, NS = info.num_cores, info.num_subcores
    NW = NC * NS
    per_w = N // NW
    CH = 128
    n_ch = per_w // CH
    mesh = plsc.VectorSubcoreMesh(core_axis_name="c", subcore_axis_name="s")

    NB = 4

    @functools.partial(
        pl.kernel,
        mesh=mesh,
        out_type=jax.ShapeDtypeStruct((N, D), jnp.float32),
        scratch_types=(
            [pltpu.VMEM((n_ch, CH), jnp.int32)]
            + [pltpu.VMEM((CH, D), jnp.float32)] * NB
            + [pltpu.SemaphoreType.DMA] * (2 * NB)
        ),
    )
    def gather(table_hbm, idx_hbm, out_hbm, idx_v, *bufs_sems):
        bufs = bufs_sems[:NB]
        gsems = bufs_sems[NB:2 * NB]
        ssems = bufs_sems[2 * NB:]
        wid = lax.axis_index("s") * NC + lax.axis_index("c")
        base = wid * per_w
        pltpu.sync_copy(idx_hbm.at[pl.ds(wid * n_ch, n_ch)], idx_v)
        gcp = [None] * n_ch
        scp = [None] * n_ch
        for j in range(n_ch):
            b = j % NB
            if j >= NB:
                scp[j - NB].wait()
            gcp[j] = pltpu.async_copy(
                table_hbm.at[idx_v.at[j]], bufs[b], gsems[b])
            if j >= 1:
                gcp[j - 1].wait()
                scp[j - 1] = pltpu.async_copy(
                    bufs[(j - 1) % NB],
                    out_hbm.at[pl.ds(base + (j - 1) * CH, CH)],
                    ssems[(j - 1) % NB])
        last = n_ch - 1
        gcp[last].wait()
        scp[last] = pltpu.async_copy(
            bufs[last % NB], out_hbm.at[pl.ds(base + last * CH, CH)],
            ssems[last % NB])
        for j in range(max(0, n_ch - NB + 1), n_ch):
            scp[j].wait()

    return gather


def _pad_body(src_ref, dst_ref):
    E = src_ref.shape[1]
    dst_ref[:, 0:E] = src_ref[...]
    dst_ref[:, E:] = jnp.zeros_like(dst_ref[:, E:])


def _pad_table(emb, EP):
    V, E = emb.shape
    RB = 20000
    return pl.pallas_call(
        _pad_body,
        grid=(V // RB,),
        in_specs=[pl.BlockSpec((RB, E), lambda i: (i, 0))],
        out_specs=pl.BlockSpec((RB, EP), lambda i: (i, 0)),
        out_shape=jax.ShapeDtypeStruct((V, EP), jnp.float32),
    )(emb)


UNROLL = 32


def _lstm_body(x_ref, wxf_ref, wxb_ref, wh_ref, bias_ref, wout_ref,
               bout_ref, out_ref, h_ref, c_ref):
    L = out_ref.shape[0]
    H2 = c_ref.shape[1]
    s = pl.program_id(0)

    @pl.when(s == 0)
    def _init():
        out_ref[...] = jnp.zeros_like(out_ref)
        h_ref[...] = jnp.zeros_like(h_ref)
        c_ref[...] = jnp.zeros_like(c_ref)

    h = h_ref[...]
    c = c_ref[...]
    T = bout_ref.shape[0]
    for k in range(UNROLL):
        t = s * UNROLL + k
        rt = (L - 1) - t
        g = (jnp.dot(x_ref[t].astype(jnp.bfloat16), wxf_ref[...],
                     preferred_element_type=jnp.float32)
             + jnp.dot(x_ref[rt].astype(jnp.bfloat16), wxb_ref[...],
                       preferred_element_type=jnp.float32)
             + bias_ref[...]
             + jnp.dot(h, wh_ref[...], preferred_element_type=jnp.float32))
        tg = jnp.tanh(g)
        gi = 0.5 * tg[:, 0:H2] + 0.5
        gf = 0.5 * tg[:, H2:2 * H2] + 0.5
        gg = tg[:, 2 * H2:3 * H2]
        go = 0.5 * tg[:, 3 * H2:4 * H2] + 0.5
        c = gf * c + gi * gg
        hf32 = go * jnp.tanh(c)
        h = hf32.astype(jnp.bfloat16)
        p = jnp.dot(hf32, wout_ref[...], preferred_element_type=jnp.float32)
        pt = jnp.swapaxes(p, 0, 1)
        out_ref[t] = out_ref[t] + pt[0:T] + bout_ref[...]
        out_ref[rt] = out_ref[rt] + pt[T:2 * T]
    h_ref[...] = h
    c_ref[...] = c


def _scatter_gates(wt, fwd):
    blocks = jnp.split(wt, 4, axis=1)
    z = jnp.zeros_like(blocks[0])
    cols = []
    for b in blocks:
        cols += ([b, z] if fwd else [z, b])
    return jnp.concatenate(cols, axis=1)


def kernel(inp, emb, w_ih_f, w_hh_f, b_ih_f, b_hh_f,
           w_ih_b, w_hh_b, b_ih_b, b_hh_b, W_out, b_out):
    B, L = inp.shape
    V, E = emb.shape
    H4 = w_ih_f.shape[0]
    H = H4 // 4
    T = W_out.shape[0]

    N = B * L
    CH = 128
    idx2 = jnp.transpose(inp).astype(jnp.int32).reshape(N // CH, CH)

    table = _pad_table(emb, EPAD)

    xg = _make_sc_gather(V, EPAD, N)(table, idx2)
    x = xg.reshape(L, B, EPAD)

    def padE(w):
        return jnp.pad(w, ((0, EPAD - E), (0, 0)))

    gscale = jnp.full((8 * H,), 0.5, jnp.float32).at[4 * H:6 * H].set(1.0)
    wxf = (_scatter_gates(padE(w_ih_f.T), True) * gscale).astype(jnp.bfloat16)
    wxb = (_scatter_gates(padE(w_ih_b.T), False) * gscale).astype(jnp.bfloat16)
    wh = (jnp.concatenate([
        _scatter_gates(w_hh_f.T, True),
        _scatter_gates(w_hh_b.T, False),
    ], axis=0) * gscale).astype(jnp.bfloat16)
    bias = ((_scatter_gates((b_ih_f + b_hh_f)[None, :], True)
             + _scatter_gates((b_ih_b + b_hh_b)[None, :], False))
            * gscale)

    wout = jnp.zeros((2 * H, 2 * T), jnp.float32)
    wout = wout.at[:H, :T].set(W_out[:, :H].T)
    wout = wout.at[H:, T:].set(W_out[:, H:].T)
    bout = b_out[:, None]

    out = pl.pallas_call(
        _lstm_body,
        grid=(L // UNROLL,),
        in_specs=[
            pl.BlockSpec((L, B, EPAD), lambda t: (0, 0, 0)),
            pl.BlockSpec((EPAD, 8 * H), lambda t: (0, 0)),
            pl.BlockSpec((EPAD, 8 * H), lambda t: (0, 0)),
            pl.BlockSpec((2 * H, 8 * H), lambda t: (0, 0)),
            pl.BlockSpec((1, 8 * H), lambda t: (0, 0)),
            pl.BlockSpec((2 * H, 2 * T), lambda t: (0, 0)),
            pl.BlockSpec((T, 1), lambda t: (0, 0)),
        ],
        out_specs=pl.BlockSpec((L, T, B), lambda t: (0, 0, 0)),
        out_shape=jax.ShapeDtypeStruct((L, T, B), jnp.float32),
        scratch_shapes=[
            pltpu.VMEM((B, 2 * H), jnp.bfloat16),
            pltpu.VMEM((B, 2 * H), jnp.float32),
        ],
        compiler_params=pltpu.CompilerParams(
            dimension_semantics=("arbitrary",)),
    )(x, wxf, wxb, wh, bias, wout, bout)

    return jnp.transpose(out, (2, 0, 1))
